# baseline (speedup 1.0000x reference)
import functools

import jax
import jax.numpy as jnp
from jax import lax
from jax.experimental import pallas as pl
from jax.experimental.pallas import tpu as pltpu
from jax.experimental.pallas import tpu_sc as plsc

L = 16
NC = 2
NS = 16
NW = NC * NS
BATCH = 16
PAIRS = 2048
IMG = 384 * 384
PPW = BATCH * PAIRS // NW
CH = PPW // L
GCH = 128
NG = PPW // GCH

_mesh = plsc.VectorSubcoreMesh(core_axis_name="c", subcore_axis_name="s")


def _loss_partials_body(img_hbm, ord_hbm, out_hbm,
                        ord_v, ia_v, ib_v, r_v, a_v, b_v, part_v, sem):
    c = lax.axis_index("c")
    s = lax.axis_index("s")
    w = s * NC + c
    base_pair = w * PPW
    img_off = (w // NC) * IMG

    pltpu.sync_copy(ord_hbm.at[pl.ds(base_pair * 5, PPW * 5)], ord_v)

    iota5 = lax.iota(jnp.int32, L) * 5

    def idx_body(j, carry):
        lanes = iota5 + j * (L * 5)
        xa = plsc.load_gather(ord_v, [lanes])
        ya = plsc.load_gather(ord_v, [lanes + 1])
        xb = plsc.load_gather(ord_v, [lanes + 2])
        yb = plsc.load_gather(ord_v, [lanes + 3])
        r = plsc.load_gather(ord_v, [lanes + 4])
        sl = pl.ds(j * L, L)
        ia_v[sl] = xa * 384 + ya + img_off
        ib_v[sl] = xb * 384 + yb + img_off
        r_v[sl] = r
        return carry

    lax.fori_loop(0, CH, idx_body, 0, unroll=True)

    copies = []
    for g in range(NG):
        gs = pl.ds(g * GCH, GCH)
        copies.append(pltpu.async_copy(img_hbm.at[ia_v.at[gs]], a_v.at[gs], sem))
        copies.append(pltpu.async_copy(img_hbm.at[ib_v.at[gs]], b_v.at[gs], sem))
    for cp in copies:
        cp.wait()

    def loss_body(j, acc):
        sl = pl.ds(j * L, L)
        a = a_v[sl]
        b = b_v[sl]
        r = r_v[sl]
        d = a - b
        t = jnp.exp(-jnp.abs(d))
        u = t / (t + 2.0)
        u2 = u * u
        poly = 1.0 + u2 * (1.0 / 3.0 + u2 * (1.0 / 5.0 + u2 * (1.0 / 7.0 + u2 * (1.0 / 9.0))))
        l1p = 2.0 * u * poly
        relu = jnp.maximum(d, 0.0)
        sp_pos = relu + l1p
        sp_neg = relu - d + l1p
        loss = jnp.where(r == 0, d * d, jnp.where(r == 1, sp_pos, sp_neg))
        return acc + loss

    acc = lax.fori_loop(0, CH, loss_body, jnp.zeros((L,), jnp.float32),
                        unroll=False)
    part_v[...] = acc
    pltpu.sync_copy(part_v, out_hbm.at[pl.ds(w * L, L)])


def _mean_body(part_hbm, out_hbm, v, o_v):
    c = lax.axis_index("c")
    s = lax.axis_index("s")
    w = s * NC + c

    @pl.when(w == 0)
    def _():
        pltpu.sync_copy(part_hbm, v)

        def body(j, acc):
            return acc + v[pl.ds(j * L, L)]

        acc = lax.fori_loop(0, NW, body, jnp.zeros((L,), jnp.float32))
        total = jnp.sum(acc)
        o_v[...] = jnp.full((L,), total * (1.0 / (BATCH * PAIRS)), jnp.float32)
        pltpu.sync_copy(o_v, out_hbm)


_partials_call = pl.kernel(
    _loss_partials_body,
    out_type=jax.ShapeDtypeStruct((NW * L,), jnp.float32),
    mesh=_mesh,
    compiler_params=pltpu.CompilerParams(needs_layout_passes=False),
    scratch_types=[
        pltpu.VMEM((PPW * 5,), jnp.int32),
        pltpu.VMEM((PPW,), jnp.int32),
        pltpu.VMEM((PPW,), jnp.int32),
        pltpu.VMEM((PPW,), jnp.int32),
        pltpu.VMEM((PPW,), jnp.float32),
        pltpu.VMEM((PPW,), jnp.float32),
        pltpu.VMEM((L,), jnp.float32),
        pltpu.SemaphoreType.DMA,
    ],
)

_mean_call = pl.kernel(
    _mean_body,
    out_type=jax.ShapeDtypeStruct((L,), jnp.float32),
    mesh=_mesh,
    compiler_params=pltpu.CompilerParams(needs_layout_passes=False),
    scratch_types=[
        pltpu.VMEM((NW * L,), jnp.float32),
        pltpu.VMEM((L,), jnp.float32),
    ],
)


@jax.jit
def kernel(output, ordinal):
    img = output.reshape(BATCH * IMG)
    ords = ordinal.astype(jnp.int32).reshape(BATCH * PAIRS * 5)
    partials = _partials_call(img, ords)
    mean_vec = _mean_call(partials)
    return mean_vec[0]


# speedup vs baseline: 1.0346x; 1.0346x over previous
import functools

import jax
import jax.numpy as jnp
from jax import lax
from jax.experimental import pallas as pl
from jax.experimental.pallas import tpu as pltpu
from jax.experimental.pallas import tpu_sc as plsc

L = 16
NC = 2
NS = 16
NW = NC * NS
BATCH = 16
PAIRS = 2048
IMG = 384 * 384
PPW = BATCH * PAIRS // NW
CH = PPW // L
GCH = 128
NG = PPW // GCH

_mesh = plsc.VectorSubcoreMesh(core_axis_name="c", subcore_axis_name="s")


def _loss_partials_body(img_hbm, ord_hbm, out_hbm,
                        ord_v, ia_v, ib_v, r_v, a_v, b_v, part_v, sem):
    c = lax.axis_index("c")
    s = lax.axis_index("s")
    w = s * NC + c
    base_pair = w * PPW
    img_off = (w // NC) * IMG

    pltpu.sync_copy(ord_hbm.at[pl.ds(base_pair * 5, PPW * 5)], ord_v)

    iota5 = lax.iota(jnp.int32, L) * 5

    def idx_body(j, carry):
        lanes = iota5 + j * (L * 5)
        xa = plsc.load_gather(ord_v, [lanes])
        ya = plsc.load_gather(ord_v, [lanes + 1])
        xb = plsc.load_gather(ord_v, [lanes + 2])
        yb = plsc.load_gather(ord_v, [lanes + 3])
        r = plsc.load_gather(ord_v, [lanes + 4])
        sl = pl.ds(j * L, L)
        ia_v[sl] = xa * 384 + ya + img_off
        ib_v[sl] = xb * 384 + yb + img_off
        r_v[sl] = r
        return carry

    lax.fori_loop(0, CH, idx_body, 0, unroll=True)

    copies = []
    for g in range(NG):
        gs = pl.ds(g * GCH, GCH)
        copies.append(pltpu.async_copy(img_hbm.at[ia_v.at[gs]], a_v.at[gs], sem))
        copies.append(pltpu.async_copy(img_hbm.at[ib_v.at[gs]], b_v.at[gs], sem))
    for cp in copies:
        cp.wait()

    def loss_body(j, acc):
        sl = pl.ds(j * L, L)
        a = a_v[sl]
        b = b_v[sl]
        r = r_v[sl]
        d = a - b
        t = jnp.exp(-jnp.abs(d))
        u = t / (t + 2.0)
        u2 = u * u
        poly = 1.0 + u2 * (1.0 / 3.0 + u2 * (1.0 / 5.0 + u2 * (1.0 / 7.0 + u2 * (1.0 / 9.0))))
        l1p = 2.0 * u * poly
        relu = jnp.maximum(d, 0.0)
        sp_pos = relu + l1p
        sp_neg = relu - d + l1p
        loss = jnp.where(r == 0, d * d, jnp.where(r == 1, sp_pos, sp_neg))
        return acc + loss

    acc = lax.fori_loop(0, CH, loss_body, jnp.zeros((L,), jnp.float32),
                        unroll=False)
    part_v[...] = acc
    pltpu.sync_copy(part_v, out_hbm.at[pl.ds(w * L, L)])


def _tc_mean_body(part_ref, out_ref):
    out_ref[...] = jnp.sum(part_ref[...], keepdims=True) * (1.0 / (BATCH * PAIRS))


_partials_call = pl.kernel(
    _loss_partials_body,
    out_type=jax.ShapeDtypeStruct((NW * L,), jnp.float32),
    mesh=_mesh,
    compiler_params=pltpu.CompilerParams(needs_layout_passes=False),
    scratch_types=[
        pltpu.VMEM((PPW * 5,), jnp.int32),
        pltpu.VMEM((PPW,), jnp.int32),
        pltpu.VMEM((PPW,), jnp.int32),
        pltpu.VMEM((PPW,), jnp.int32),
        pltpu.VMEM((PPW,), jnp.float32),
        pltpu.VMEM((PPW,), jnp.float32),
        pltpu.VMEM((L,), jnp.float32),
        pltpu.SemaphoreType.DMA,
    ],
)

_tc_mean_call = pl.pallas_call(
    _tc_mean_body,
    out_shape=jax.ShapeDtypeStruct((1, 1), jnp.float32),
)


@jax.jit
def kernel(output, ordinal):
    img = output.reshape(BATCH * IMG)
    ords = ordinal.astype(jnp.int32).reshape(BATCH * PAIRS * 5)
    partials = _partials_call(img, ords)
    return _tc_mean_call(partials.reshape(NW, L))[0, 0]


# speedup vs baseline: 1.6796x; 1.6235x over previous
import functools

import jax
import jax.numpy as jnp
from jax import lax
from jax.experimental import pallas as pl
from jax.experimental.pallas import tpu as pltpu
from jax.experimental.pallas import tpu_sc as plsc

L = 16
NC = 2
NS = 16
NW = NC * NS
BATCH = 16
PAIRS = 2048
IMG = 384 * 384
PPW = BATCH * PAIRS // NW
CH = PPW // L
GCH = 128
NG = PPW // GCH

_mesh = plsc.VectorSubcoreMesh(core_axis_name="c", subcore_axis_name="s")


def _loss_partials_body(img_hbm, ord_hbm, out_hbm,
                        xa_v, ya_v, xb_v, yb_v, r_v,
                        ia_v, ib_v, a_v, b_v, part_v, sem):
    c = lax.axis_index("c")
    s = lax.axis_index("s")
    w = s * NC + c
    n = w // NC
    h = w % NC
    img_off = n * IMG

    psl = pl.ds(h * PPW, PPW)
    pltpu.sync_copy(ord_hbm.at[0, n, psl], xa_v)
    pltpu.sync_copy(ord_hbm.at[1, n, psl], ya_v)
    pltpu.sync_copy(ord_hbm.at[2, n, psl], xb_v)
    pltpu.sync_copy(ord_hbm.at[3, n, psl], yb_v)
    pltpu.sync_copy(ord_hbm.at[4, n, psl], r_v)

    def idx_body(j, carry):
        sl = pl.ds(j * L, L)
        ia_v[sl] = xa_v[sl] * 384 + ya_v[sl] + img_off
        ib_v[sl] = xb_v[sl] * 384 + yb_v[sl] + img_off
        return carry

    lax.fori_loop(0, CH, idx_body, 0, unroll=True)

    copies = []
    for g in range(NG):
        gs = pl.ds(g * GCH, GCH)
        copies.append(pltpu.async_copy(img_hbm.at[ia_v.at[gs]], a_v.at[gs], sem))
        copies.append(pltpu.async_copy(img_hbm.at[ib_v.at[gs]], b_v.at[gs], sem))
    for cp in copies:
        cp.wait()

    def loss_body(j, acc):
        sl = pl.ds(j * L, L)
        a = a_v[sl]
        b = b_v[sl]
        r = r_v[sl]
        d = a - b
        t = jnp.exp(-jnp.abs(d))
        u = t / (t + 2.0)
        u2 = u * u
        poly = 1.0 + u2 * (1.0 / 3.0 + u2 * (1.0 / 5.0 + u2 * (1.0 / 7.0 + u2 * (1.0 / 9.0))))
        l1p = 2.0 * u * poly
        relu = jnp.maximum(d, 0.0)
        sp_pos = relu + l1p
        sp_neg = relu - d + l1p
        loss = jnp.where(r == 0, d * d, jnp.where(r == 1, sp_pos, sp_neg))
        return acc + loss

    acc = lax.fori_loop(0, CH, loss_body, jnp.zeros((L,), jnp.float32),
                        unroll=False)
    part_v[...] = acc
    pltpu.sync_copy(part_v, out_hbm.at[pl.ds(w * L, L)])


def _tc_mean_body(part_ref, out_ref):
    out_ref[...] = jnp.sum(part_ref[...], keepdims=True) * (1.0 / (BATCH * PAIRS))


_partials_call = pl.kernel(
    _loss_partials_body,
    out_type=jax.ShapeDtypeStruct((NW * L,), jnp.float32),
    mesh=_mesh,
    compiler_params=pltpu.CompilerParams(needs_layout_passes=False),
    scratch_types=[
        pltpu.VMEM((PPW,), jnp.int32),
        pltpu.VMEM((PPW,), jnp.int32),
        pltpu.VMEM((PPW,), jnp.int32),
        pltpu.VMEM((PPW,), jnp.int32),
        pltpu.VMEM((PPW,), jnp.int32),
        pltpu.VMEM((PPW,), jnp.int32),
        pltpu.VMEM((PPW,), jnp.int32),
        pltpu.VMEM((PPW,), jnp.float32),
        pltpu.VMEM((PPW,), jnp.float32),
        pltpu.VMEM((L,), jnp.float32),
        pltpu.SemaphoreType.DMA,
    ],
)

_tc_mean_call = pl.pallas_call(
    _tc_mean_body,
    out_shape=jax.ShapeDtypeStruct((1, 1), jnp.float32),
)


@jax.jit
def kernel(output, ordinal):
    img = output.reshape(BATCH * IMG)
    planes = jnp.transpose(ordinal.astype(jnp.int32), (2, 0, 1))
    partials = _partials_call(img, planes)
    return _tc_mean_call(partials.reshape(NW, L))[0, 0]
